# Initial kernel scaffold; baseline (speedup 1.0000x reference)
import jax
import jax.numpy as jnp
from jax.experimental import pallas as pl


def kernel(confidence, predicted_locations, labels, gt_locations):
    raise NotImplementedError("write your pallas kernel here")


# speedup vs baseline: 18.7552x; 18.7552x over previous
import functools

import jax
import jax.numpy as jnp
import numpy as np
from jax import lax
from jax.experimental import pallas as pl
from jax.experimental.pallas import tpu as pltpu

_B, _P, _C = 32, 20000, 3
_MSB = np.int32(-2147483648)


def _row_kernel(c0_ref, c1_ref, c2_ref, lab_ref, pd_ref, gt_ref,
                out_sl1_ref, out_cls_ref,
                ukey_s, mine_s, npos_s, pnll_s, sl1_s):
    i = pl.program_id(0)

    c0 = c0_ref[0, 0, :]
    c1 = c1_ref[0, 0, :]
    c2 = c2_ref[0, 0, :]
    lab = lab_ref[0, 0, :]

    m = jnp.maximum(c0, jnp.maximum(c1, c2))
    lse = m + jnp.log(jnp.exp(c0 - m) + jnp.exp(c1 - m) + jnp.exp(c2 - m))
    mining = lse - c0
    pos = lab > 0

    bits = lax.bitcast_convert_type(mining, jnp.int32)
    u = jnp.where(bits >= 0, bits | _MSB, ~bits)
    u = jnp.where(pos, np.int32(0), u)

    ukey_s[pl.ds(i, 1), :] = u.reshape(1, _P)
    mine_s[pl.ds(i, 1), :] = mining.reshape(1, _P)

    npos = jnp.sum(pos.astype(jnp.int32))
    sel = jnp.where(lab == 1, c1, c2)
    pnll = jnp.sum(jnp.where(pos, lse - sel, 0.0))

    d = pd_ref[0] - gt_ref[0]
    ad = jnp.abs(d)
    sl1 = jnp.where(ad < 1.0, 0.5 * d * d, ad - 0.5)
    sl1_row = jnp.sum(jnp.where(pos[None, :], sl1, 0.0))

    npos_s[pl.ds(i, 1), :] = jnp.full((1, 128), npos, jnp.int32)
    pnll_s[pl.ds(i, 1), :] = jnp.full((1, 128), pnll, jnp.float32)
    sl1_s[pl.ds(i, 1), :] = jnp.full((1, 128), sl1_row, jnp.float32)

    @pl.when(i == _B - 1)
    def _finalize():
        U = ukey_s[...]
        M = mine_s[...]
        npv = npos_s[:, 0:1]
        k = jnp.minimum(3 * npv, _P - npv)

        ux = U ^ _MSB
        t = jnp.zeros((_B, 1), jnp.int32)
        for bit in range(31, -1, -1):
            cand = t | np.int32(np.uint32(1 << bit))
            cnt = jnp.sum((ux >= (cand ^ _MSB)).astype(jnp.int32),
                          axis=1, keepdims=True)
            t = jnp.where(cnt >= k, cand, t)

        gt_m = ux > (t ^ _MSB)
        cnt_gt = jnp.sum(gt_m.astype(jnp.int32), axis=1, keepdims=True)
        sum_gt = jnp.sum(jnp.where(gt_m, M, 0.0), axis=1, keepdims=True)
        tbits = jnp.where(t < 0, t & np.int32(0x7FFFFFFF), ~t)
        tval = lax.bitcast_convert_type(tbits, jnp.float32)
        neg_row = jnp.where(k > 0,
                            sum_gt + (k - cnt_gt).astype(jnp.float32) * tval,
                            0.0)

        np_tot = jnp.sum(npv).astype(jnp.float32)
        cls_tot = jnp.sum(pnll_s[:, 0:1]) + 0.2 * jnp.sum(neg_row)
        sl1_tot = jnp.sum(sl1_s[:, 0:1])
        out_sl1_ref[...] = (sl1_tot / np_tot).reshape(1, 1)
        out_cls_ref[...] = (cls_tot / np_tot).reshape(1, 1)


@jax.jit
def kernel(confidence, predicted_locations, labels, gt_locations):
    B, P, C = confidence.shape
    c0 = confidence[:, :, 0].reshape(B, 1, P)
    c1 = confidence[:, :, 1].reshape(B, 1, P)
    c2 = confidence[:, :, 2].reshape(B, 1, P)
    lab = labels.reshape(B, 1, P)
    pdT = predicted_locations.transpose(0, 2, 1)
    gtT = gt_locations.transpose(0, 2, 1)

    row_spec = pl.BlockSpec((1, 1, P), lambda i: (i, 0, 0))
    loc_spec = pl.BlockSpec((1, 4, P), lambda i: (i, 0, 0))
    out_spec = pl.BlockSpec((1, 1), lambda i: (0, 0))

    out_sl1, out_cls = pl.pallas_call(
        _row_kernel,
        grid=(B,),
        in_specs=[row_spec, row_spec, row_spec, row_spec, loc_spec, loc_spec],
        out_specs=[out_spec, out_spec],
        out_shape=[jax.ShapeDtypeStruct((1, 1), jnp.float32),
                   jax.ShapeDtypeStruct((1, 1), jnp.float32)],
        scratch_shapes=[
            pltpu.VMEM((B, P), jnp.int32),
            pltpu.VMEM((B, P), jnp.float32),
            pltpu.VMEM((B, 128), jnp.int32),
            pltpu.VMEM((B, 128), jnp.float32),
            pltpu.VMEM((B, 128), jnp.float32),
        ],
    )(c0, c1, c2, lab, pdT, gtT)
    return (out_sl1[0, 0], out_cls[0, 0])
